# Initial kernel scaffold; baseline (speedup 1.0000x reference)
import jax
import jax.numpy as jnp
from jax.experimental import pallas as pl


def kernel(x, edge_index, edge_attr, batch, Wq1, bq1, Wk1, bk1, Wv1, bv1, We1, Ws1, bs1, Wq2, bq2, Wk2, bk2, Wv2, bv2, We2, Ws2, bs2, Wc1, bc1, Wc2, bc2):
    raise NotImplementedError("write your pallas kernel here")


# speedup vs baseline: 32.8091x; 32.8091x over previous
import functools

import jax
import jax.numpy as jnp
from jax import lax
from jax.experimental import pallas as pl
from jax.experimental.pallas import tpu as pltpu
from jax.experimental.pallas import tpu_sc as plsc

_N = 10000
_E = 320000
_ED = 16
_HID = 64
_H = 2
_NCLS = 30
_G = 16
_HD = _H * _HID

_K = 128
_NCHUNK = _E // _K
_NWORK = 32
_CHUNKS_PER_W = -(-_NCHUNK // _NWORK)
_DA = 48
_NPAD = 10240
_ROWS_PER_TILE = _NPAD // 16

_SC_PARAMS = pltpu.CompilerParams(
    use_tc_tiling_on_sc=False, needs_layout_passes=False)
_MESH = plsc.VectorSubcoreMesh(core_axis_name="c", subcore_axis_name="s")


def _proj_body(x_ref, wq, bq, wk, bk, wv, bv, we, ws, bs,
               k_ref, v_ref, qq_ref, skip_ref):
    x = x_ref[...]
    q = jnp.dot(x, wq[...].T, preferred_element_type=jnp.float32) + bq[...]
    k_ref[...] = jnp.dot(x, wk[...].T, preferred_element_type=jnp.float32) + bk[...]
    v_ref[...] = jnp.dot(x, wv[...].T, preferred_element_type=jnp.float32) + bv[...]
    qq_ref[:, :_HD] = q
    we_v = we[...]
    qq_ref[:, _HD:_HD + _ED] = jnp.dot(q[:, :_HID], we_v[:_HID, :],
                                       preferred_element_type=jnp.float32)
    qq_ref[:, _HD + _ED:] = jnp.dot(q[:, _HID:], we_v[_HID:, :],
                                    preferred_element_type=jnp.float32)
    skip_ref[...] = jnp.dot(x, ws[...].T, preferred_element_type=jnp.float32) + bs[...]


def _proj_call(x, wq, bq, wk, bk, wv, bv, we, ws, bs):
    blk = 2000
    row_spec = lambda w: pl.BlockSpec((blk, w), lambda i: (i, 0))
    full = lambda a: pl.BlockSpec(a.shape, lambda i: tuple(0 for _ in a.shape))
    return pl.pallas_call(
        _proj_body,
        grid=(_N // blk,),
        in_specs=[pl.BlockSpec((blk, x.shape[1]), lambda i: (i, 0)),
                  full(wq), full(bq), full(wk), full(bk), full(wv), full(bv),
                  full(we), full(ws), full(bs)],
        out_specs=[row_spec(_HD), row_spec(_HD), row_spec(_HD + 2 * _ED),
                   row_spec(_HD)],
        out_shape=[jax.ShapeDtypeStruct((_N, _HD), jnp.float32),
                   jax.ShapeDtypeStruct((_N, _HD), jnp.float32),
                   jax.ShapeDtypeStruct((_N, _HD + 2 * _ED), jnp.float32),
                   jax.ShapeDtypeStruct((_N, _HD), jnp.float32)],
    )(x, wq, bq, wk, bk, wv, bv, we, ws, bs)


def _zero_rows(buf, width):
    zero16 = jnp.zeros((16,), jnp.float32)

    def _zrow(r, carry):
        for c in range(width // 16):
            buf[r, pl.ds(c * 16, 16)] = zero16
        return carry
    lax.fori_loop(0, _K, _zrow, 0)


def _alpha_sc_body(src_hbm, dst_hbm, ea_hbm, k_hbm, qq_hbm,
                   acc_hbm, ex_hbm,
                   src_v, dst_v, k_v, qq_v, ea_v, orow_v, ex_v, acc_sh,
                   sem1, sem2, sem3):
    cid = lax.axis_index("c")
    sid = lax.axis_index("s")
    wid = sid * 2 + cid

    zero16 = jnp.zeros((16,), jnp.float32)
    lane = jnp.arange(16, dtype=jnp.int32)

    _zero_rows(orow_v, _DA)
    for i in range(5):
        pltpu.sync_copy(orow_v,
                        acc_sh.at[pl.ds(sid * _ROWS_PER_TILE + i * _K, _K)])
    plsc.subcore_barrier()

    def _chunk(i, carry):
        c = i * _NWORK + wid

        @pl.when(c < _NCHUNK)
        def _():
            off = c * _K
            pltpu.sync_copy(src_hbm.at[pl.ds(off, _K)], src_v)
            pltpu.sync_copy(dst_hbm.at[pl.ds(off, _K)], dst_v)
            cp1 = pltpu.async_copy(k_hbm.at[src_v], k_v, sem1)
            cp2 = pltpu.async_copy(qq_hbm.at[dst_v], qq_v, sem2)
            cp3 = pltpu.async_copy(ea_hbm.at[pl.ds(off, _K)], ea_v, sem3)
            cp1.wait()
            cp2.wait()
            cp3.wait()

            def _edge(j, exvec):
                ea = ea_v[j, :]
                s0 = qq_v[j, pl.ds(0, 16)] * k_v[j, pl.ds(0, 16)]
                for t in range(1, 4):
                    s0 = s0 + qq_v[j, pl.ds(t * 16, 16)] * k_v[j, pl.ds(t * 16, 16)]
                s0 = s0 + qq_v[j, pl.ds(128, 16)] * ea
                ex0 = jnp.exp(jnp.broadcast_to(jnp.sum(s0) * 0.125, (16,)))

                s1 = qq_v[j, pl.ds(64, 16)] * k_v[j, pl.ds(64, 16)]
                for t in range(5, 8):
                    s1 = s1 + qq_v[j, pl.ds(t * 16, 16)] * k_v[j, pl.ds(t * 16, 16)]
                s1 = s1 + qq_v[j, pl.ds(144, 16)] * ea
                ex1 = jnp.exp(jnp.broadcast_to(jnp.sum(s1) * 0.125, (16,)))

                orow_v[j, pl.ds(0, 16)] = ex0 * ea
                orow_v[j, pl.ds(16, 16)] = ex1 * ea
                exl = jnp.where(lane == 0, ex0, jnp.where(lane == 1, ex1, zero16))
                orow_v[j, pl.ds(32, 16)] = exl

                p = lax.bitwise_and(j, 7) * 2
                base = jnp.where(lax.bitwise_and(j, 7) == 0, zero16, exvec)
                exvec = jnp.where(lane == p, ex0,
                                  jnp.where(lane == p + 1, ex1, base))
                ex_v[pl.ds(lax.div(j, 8) * 16, 16)] = exvec
                return exvec

            lax.fori_loop(0, _K, _edge, zero16)
            pltpu.sync_copy(orow_v, acc_sh.at[dst_v], add=True)
            pltpu.sync_copy(ex_v, ex_hbm.at[pl.ds(off * 2, 2 * _K)])
        return carry

    lax.fori_loop(0, _CHUNKS_PER_W, _chunk, 0)
    plsc.subcore_barrier()
    for i in range(5):
        rows = pl.ds(sid * _ROWS_PER_TILE + i * _K, _K)
        pltpu.sync_copy(acc_sh.at[rows], acc_hbm.at[cid, rows])


def _value_sc_body(src_hbm, dst_hbm, ex_hbm, v_hbm,
                   acc_hbm,
                   src_v, dst_v, v_v, ex_v, orow_v, acc_sh,
                   sem1, sem2):
    cid = lax.axis_index("c")
    sid = lax.axis_index("s")
    wid = sid * 2 + cid

    _zero_rows(orow_v, _HD)
    for i in range(5):
        pltpu.sync_copy(orow_v,
                        acc_sh.at[pl.ds(sid * _ROWS_PER_TILE + i * _K, _K)])
    plsc.subcore_barrier()

    def _chunk(i, carry):
        c = i * _NWORK + wid

        @pl.when(c < _NCHUNK)
        def _():
            off = c * _K
            pltpu.sync_copy(src_hbm.at[pl.ds(off, _K)], src_v)
            pltpu.sync_copy(dst_hbm.at[pl.ds(off, _K)], dst_v)
            cp1 = pltpu.async_copy(v_hbm.at[src_v], v_v, sem1)
            cp2 = pltpu.async_copy(ex_hbm.at[pl.ds(off * 2, 2 * _K)],
                                   ex_v.at[pl.ds(0, 2 * _K)], sem2)
            cp1.wait()
            cp2.wait()

            def _edge(j, carry2):
                exv = ex_v[pl.ds(2 * j, 16)]
                ex0 = jnp.broadcast_to(exv[0], (16,))
                ex1 = jnp.broadcast_to(exv[1], (16,))
                for t in range(4):
                    orow_v[j, pl.ds(t * 16, 16)] = ex0 * v_v[j, pl.ds(t * 16, 16)]
                for t in range(4, 8):
                    orow_v[j, pl.ds(t * 16, 16)] = ex1 * v_v[j, pl.ds(t * 16, 16)]
                return carry2

            lax.fori_loop(0, _K, _edge, 0)
            pltpu.sync_copy(orow_v, acc_sh.at[dst_v], add=True)
        return carry

    lax.fori_loop(0, _CHUNKS_PER_W, _chunk, 0)
    plsc.subcore_barrier()
    for i in range(5):
        rows = pl.ds(sid * _ROWS_PER_TILE + i * _K, _K)
        pltpu.sync_copy(acc_sh.at[rows], acc_hbm.at[cid, rows])


_alpha_call = functools.partial(
    pl.kernel,
    mesh=_MESH,
    compiler_params=_SC_PARAMS,
    out_type=[jax.ShapeDtypeStruct((2, _NPAD, _DA), jnp.float32),
              jax.ShapeDtypeStruct((2 * _E,), jnp.float32)],
    scratch_types=[
        pltpu.VMEM((_K,), jnp.int32),
        pltpu.VMEM((_K,), jnp.int32),
        pltpu.VMEM((_K, _HD), jnp.float32),
        pltpu.VMEM((_K, _HD + 2 * _ED), jnp.float32),
        pltpu.VMEM((_K, _ED), jnp.float32),
        pltpu.VMEM((_K, _DA), jnp.float32),
        pltpu.VMEM((2 * _K,), jnp.float32),
        pltpu.VMEM_SHARED((_NPAD, _DA), jnp.float32),
        pltpu.SemaphoreType.DMA,
        pltpu.SemaphoreType.DMA,
        pltpu.SemaphoreType.DMA,
    ],
)(_alpha_sc_body)

_value_call = functools.partial(
    pl.kernel,
    mesh=_MESH,
    compiler_params=_SC_PARAMS,
    out_type=jax.ShapeDtypeStruct((2, _NPAD, _HD), jnp.float32),
    scratch_types=[
        pltpu.VMEM((_K,), jnp.int32),
        pltpu.VMEM((_K,), jnp.int32),
        pltpu.VMEM((_K, _HD), jnp.float32),
        pltpu.VMEM((2 * _K + 16,), jnp.float32),
        pltpu.VMEM((_K, _HD), jnp.float32),
        pltpu.VMEM_SHARED((_NPAD, _HD), jnp.float32),
        pltpu.SemaphoreType.DMA,
        pltpu.SemaphoreType.DMA,
    ],
)(_value_sc_body)


def _norm_h(acca, accv, we_v, skip):
    den0 = acca[:, 32:33] + 1e-16
    den1 = acca[:, 33:34] + 1e-16
    out0 = (accv[:, :_HID] +
            jnp.dot(acca[:, 0:16], we_v[:_HID, :].T,
                    preferred_element_type=jnp.float32)) / den0
    out1 = (accv[:, _HID:] +
            jnp.dot(acca[:, 16:32], we_v[_HID:, :].T,
                    preferred_element_type=jnp.float32)) / den1
    h = jnp.concatenate([out0, out1], axis=1) + skip
    return jnp.maximum(h, 0.0)


def _post_body(acca_ref, accv_ref, skip_ref, we_ref, h_ref):
    acca = acca_ref[0] + acca_ref[1]
    accv = accv_ref[0] + accv_ref[1]
    h_ref[...] = _norm_h(acca, accv, we_ref[...], skip_ref[...])


def _post_call(acca, accv, skip, we):
    blk = 2000
    return pl.pallas_call(
        _post_body,
        grid=(_N // blk,),
        in_specs=[pl.BlockSpec((2, blk, _DA), lambda i: (0, i, 0)),
                  pl.BlockSpec((2, blk, _HD), lambda i: (0, i, 0)),
                  pl.BlockSpec((blk, _HD), lambda i: (i, 0)),
                  pl.BlockSpec(we.shape, lambda i: (0, 0))],
        out_specs=pl.BlockSpec((blk, _HD), lambda i: (i, 0)),
        out_shape=jax.ShapeDtypeStruct((_N, _HD), jnp.float32),
    )(acca, accv, skip, we)


def _head_body(acca_ref, accv_ref, skip_ref, we_ref, batch_ref,
               wc1, bc1, wc2, bc2, out_ref):
    acca = acca_ref[0] + acca_ref[1]
    accv = accv_ref[0] + accv_ref[1]
    h = _norm_h(acca, accv, we_ref[...], skip_ref[...])

    b = batch_ref[...]
    gid = lax.broadcasted_iota(jnp.int32, (_G, _N), 0)
    m = jnp.where(b == gid, 1.0, 0.0)
    sums = jnp.dot(m, h, preferred_element_type=jnp.float32)
    cnt = jnp.sum(m, axis=1, keepdims=True)
    pooled = sums / jnp.maximum(cnt, 1.0)
    hid = jnp.maximum(jnp.dot(pooled, wc1[...].T,
                              preferred_element_type=jnp.float32) + bc1[...], 0.0)
    out_ref[...] = jnp.dot(hid, wc2[...].T,
                           preferred_element_type=jnp.float32) + bc2[...]


def _head_call(acca, accv, skip, we, batch2d, wc1, bc1, wc2, bc2):
    return pl.pallas_call(
        _head_body,
        out_shape=jax.ShapeDtypeStruct((_G, _NCLS), jnp.float32),
    )(acca, accv, skip, we, batch2d, wc1, bc1, wc2, bc2)


def _layer(h, src, dst, edge_attr, wq, bq, wk, bk, wv, bv, we, ws, bs):
    k, v, qq, skip = _proj_call(h, wq, bq, wk, bk, wv, bv, we, ws, bs)
    acca, ex = _alpha_call(src, dst, edge_attr, k, qq)
    accv = _value_call(src, dst, ex, v)
    return acca[:, :_N], accv[:, :_N], skip


def kernel(x, edge_index, edge_attr, batch,
           Wq1, bq1, Wk1, bk1, Wv1, bv1, We1, Ws1, bs1,
           Wq2, bq2, Wk2, bk2, Wv2, bv2, We2, Ws2, bs2,
           Wc1, bc1, Wc2, bc2):
    src = edge_index[0].astype(jnp.int32)
    dst = edge_index[1].astype(jnp.int32)
    r2 = lambda b: b.reshape(1, -1)

    acca1, accv1, skip1 = _layer(x, src, dst, edge_attr,
                                 Wq1, r2(bq1), Wk1, r2(bk1), Wv1, r2(bv1),
                                 We1, Ws1, r2(bs1))
    h1 = _post_call(acca1, accv1, skip1, We1)
    acca2, accv2, skip2 = _layer(h1, src, dst, edge_attr,
                                 Wq2, r2(bq2), Wk2, r2(bk2), Wv2, r2(bv2),
                                 We2, Ws2, r2(bs2))
    return _head_call(acca2, accv2, skip2, We2,
                      batch.astype(jnp.int32).reshape(1, -1),
                      Wc1, r2(bc1), Wc2, r2(bc2))
